# baseline (speedup 1.0000x reference)
import functools

import jax
import jax.numpy as jnp
from jax import lax
from jax.experimental import pallas as pl
from jax.experimental.pallas import tpu as pltpu
from jax.experimental.pallas import tpu_sc as plsc

N = 320000
C = 128
NUM_CORES = 2
NUM_SUBCORES = 16
NW = NUM_CORES * NUM_SUBCORES
ROWS_PER_W = N // NW
R = 80
NCHUNK = ROWS_PER_W // R
VPR = C // 16

_mesh = plsc.VectorSubcoreMesh(core_axis_name="c", subcore_axis_name="s")


@functools.partial(
    pl.kernel,
    out_type=jax.ShapeDtypeStruct((N, C), jnp.float32),
    mesh=_mesh,
    scratch_types=[
        pltpu.VMEM((R,), jnp.int32),
        pltpu.VMEM((R, C), jnp.float32),
        pltpu.VMEM((R, C), jnp.float32),
        pltpu.SemaphoreType.DMA,
    ],
)
def _broadcast_add(feat_hbm, glob_hbm, ids_hbm, out_hbm, ids_v, in_v, gat_v, sem):
    wid = lax.axis_index("s") * NUM_CORES + lax.axis_index("c")
    base = wid * ROWS_PER_W

    def chunk_body(ci, carry):
        off = base + ci * R
        pltpu.sync_copy(ids_hbm.at[pl.ds(off, R)], ids_v)
        cp_in = pltpu.async_copy(feat_hbm.at[pl.ds(off, R)], in_v, sem)
        cp_gat = pltpu.async_copy(glob_hbm.at[ids_v], gat_v, sem)
        cp_in.wait()
        cp_gat.wait()

        def row_body(r, c2):
            for k in range(VPR):
                sl = pl.ds(k * 16, 16)
                in_v[r, sl] = in_v[r, sl] + gat_v[r, sl]
            return c2

        lax.fori_loop(0, R, row_body, 0)
        pltpu.sync_copy(in_v, out_hbm.at[pl.ds(off, R)])
        return carry

    lax.fori_loop(0, NCHUNK, chunk_body, 0)


def kernel(input_features, input_features_global, batch_ids):
    ids = batch_ids.astype(jnp.int32)
    return _broadcast_add(input_features, input_features_global, ids)


# speedup vs baseline: 9.1646x; 9.1646x over previous
import functools

import jax
import jax.numpy as jnp
from jax import lax
from jax.experimental import pallas as pl
from jax.experimental.pallas import tpu as pltpu
from jax.experimental.pallas import tpu_sc as plsc

N = 320000
C = 128
B = 16
NUM_CORES = 2
NUM_SUBCORES = 16
NW = NUM_CORES * NUM_SUBCORES
ROWS_PER_W = N // NW
R = 200
NCHUNK = ROWS_PER_W // R
NBUF = 2
NGRP = NCHUNK // NBUF
VPR = C // 16

_mesh = plsc.VectorSubcoreMesh(core_axis_name="c", subcore_axis_name="s")


@functools.partial(
    pl.kernel,
    out_type=jax.ShapeDtypeStruct((N, C), jnp.float32),
    mesh=_mesh,
    scratch_types=[
        pltpu.VMEM((B, C), jnp.float32),
        pltpu.VMEM((R, C), jnp.float32),
        pltpu.VMEM((R, C), jnp.float32),
        pltpu.VMEM((R, C), jnp.float32),
        pltpu.VMEM((R, C), jnp.float32),
        pltpu.VMEM((R + 16,), jnp.int32),
        pltpu.VMEM((R + 16,), jnp.int32),
        pltpu.SemaphoreType.DMA,
        pltpu.SemaphoreType.DMA,
        pltpu.SemaphoreType.DMA,
        pltpu.SemaphoreType.DMA,
    ],
)
def _broadcast_add(feat_hbm, glob_hbm, ids_hbm, out_hbm, table_v,
                   in0, in1, out0, out1, ids0, ids1,
                   isem0, isem1, osem0, osem1):
    wid = lax.axis_index("s") * NUM_CORES + lax.axis_index("c")
    base = wid * ROWS_PER_W
    in_bufs = (in0, in1)
    out_bufs = (out0, out1)
    ids_bufs = (ids0, ids1)
    in_sems = (isem0, isem1)
    out_sems = (osem0, osem1)

    pltpu.sync_copy(glob_hbm, table_v)

    def start_in(j, b):
        off = base + j * R
        pltpu.async_copy(ids_hbm.at[pl.ds(off, R)], ids_bufs[b].at[pl.ds(0, R)],
                         in_sems[b])
        pltpu.async_copy(feat_hbm.at[pl.ds(off, R)], in_bufs[b], in_sems[b])

    def wait_in(b):
        pltpu.make_async_copy(ids_hbm.at[pl.ds(0, R)], ids_bufs[b].at[pl.ds(0, R)],
                              in_sems[b]).wait()
        pltpu.make_async_copy(feat_hbm.at[pl.ds(0, R)], in_bufs[b], in_sems[b]).wait()

    def wait_out(b):
        pltpu.make_async_copy(out_hbm.at[pl.ds(0, R)], out_bufs[b], out_sems[b]).wait()

    def compute(b):
        in_v, out_v, ids_v = in_bufs[b], out_bufs[b], ids_bufs[b]
        first = ids_v[pl.ds(0, 16)][0]
        last = ids_v[pl.ds(R - 16, 16)][15]

        def fast():
            tab = [table_v[first, pl.ds(k * 16, 16)] for k in range(VPR)]

            def row_body(r, c2):
                for k in range(VPR):
                    sl = pl.ds(k * 16, 16)
                    out_v[r, sl] = in_v[r, sl] + tab[k]
                return c2

            lax.fori_loop(0, R, row_body, 0)

        def slow():
            def row_body(r, c2):
                rid = ids_v[pl.ds(r, 16)][0]
                for k in range(VPR):
                    sl = pl.ds(k * 16, 16)
                    out_v[r, sl] = in_v[r, sl] + table_v[rid, sl]
                return c2

            lax.fori_loop(0, R, row_body, 0)

        lax.cond(first == last, fast, slow)

    for b in range(NBUF):
        start_in(b, b)

    def group_body(g, carry):
        for b in range(NBUF):
            j = g * NBUF + b
            wait_in(b)

            @pl.when(g >= 1)
            def _():
                wait_out(b)

            compute(b)
            off = base + j * R
            pltpu.async_copy(out_bufs[b], out_hbm.at[pl.ds(off, R)], out_sems[b])

            @pl.when(g < NGRP - 1)
            def _():
                start_in(j + NBUF, b)

        return carry

    lax.fori_loop(0, NGRP, group_body, 0)
    for b in range(NBUF):
        wait_out(b)


def kernel(input_features, input_features_global, batch_ids):
    ids = batch_ids.astype(jnp.int32)
    return _broadcast_add(input_features, input_features_global, ids)
